# baseline (speedup 1.0000x reference)
import functools

import jax
import jax.numpy as jnp
from jax import lax
from jax.experimental import pallas as pl
from jax.experimental.pallas import tpu as pltpu
from jax.experimental.pallas import tpu_sc as plsc

NUM_CAMERAS = 100000
BATCH = 16384

_info = plsc.get_sparse_core_info()
_NC, _NS = _info.num_cores, _info.num_subcores
_NW = _NC * _NS
_BPW = BATCH // _NW
_CHUNK = 128
_NCHUNK = _BPW // _CHUNK


def _make_gather(nrow, npad):
    @functools.partial(
        pl.kernel,
        mesh=plsc.VectorSubcoreMesh(core_axis_name="c", subcore_axis_name="s"),
        out_type=jax.ShapeDtypeStruct((BATCH // _CHUNK, npad, _CHUNK), jnp.float32),
        scratch_types=[
            pltpu.VMEM((_NCHUNK, _CHUNK), jnp.int32),
            pltpu.VMEM((nrow, _BPW), jnp.float32),
            pltpu.SemaphoreType.DMA,
        ],
        compiler_params=pltpu.CompilerParams(use_tc_tiling_on_sc=False),
    )
    def _k(tbl_hbm, idx_hbm, out, idx_v, val_v, sem):
        wid = lax.axis_index("s") * _NC + lax.axis_index("c")
        pltpu.sync_copy(idx_hbm.at[pl.ds(wid * _NCHUNK, _NCHUNK)], idx_v)
        copies = []
        for j in range(_NCHUNK):
            sl = pl.ds(j * _CHUNK, _CHUNK)
            for r in range(nrow):
                copies.append(pltpu.async_copy(
                    tbl_hbm.at[r].at[idx_v.at[j]], val_v.at[r, sl], sem))
        for c in copies:
            c.wait()
        for j in range(_NCHUNK):
            gc = wid * _NCHUNK + j
            sl = pl.ds(j * _CHUNK, _CHUNK)
            pltpu.sync_copy(val_v.at[:, sl], out.at[gc, pl.ds(0, nrow)])
    return _k


_gather_pos = _make_gather(3, 4)
_gather_rot = _make_gather(6, 8)


def kernel(positions, rotations, camera_indices):
    idx2d = camera_indices.astype(jnp.int32).reshape(BATCH // _CHUNK, _CHUNK)
    pos3 = _gather_pos(positions.T, idx2d)
    rot3 = _gather_rot(rotations.T, idx2d)
    pos = pos3.transpose(0, 2, 1).reshape(BATCH, 4)[:, :3]
    rot = rot3.transpose(0, 2, 1).reshape(BATCH, 8)[:, :6]
    return pos, rot


# speedup vs baseline: 1.0915x; 1.0915x over previous
import functools

import jax
import jax.numpy as jnp
from jax import lax
from jax.experimental import pallas as pl
from jax.experimental.pallas import tpu as pltpu
from jax.experimental.pallas import tpu_sc as plsc

NUM_CAMERAS = 100000
BATCH = 16384

_info = plsc.get_sparse_core_info()
_NC, _NS = _info.num_cores, _info.num_subcores
_NW = _NC * _NS
_BPW = BATCH // _NW
_CHUNK = 128
_NCHUNK = _BPW // _CHUNK


@functools.partial(
    pl.kernel,
    mesh=plsc.VectorSubcoreMesh(core_axis_name="c", subcore_axis_name="s"),
    out_type=(
        jax.ShapeDtypeStruct((BATCH // _CHUNK, 4, _CHUNK), jnp.float32),
        jax.ShapeDtypeStruct((BATCH // _CHUNK, 8, _CHUNK), jnp.float32),
    ),
    scratch_types=[
        pltpu.VMEM((_NCHUNK, _CHUNK), jnp.int32),
        pltpu.VMEM((_NCHUNK, 3, _CHUNK), jnp.float32),
        pltpu.VMEM((_NCHUNK, 6, _CHUNK), jnp.float32),
        pltpu.SemaphoreType.DMA,
        pltpu.SemaphoreType.DMA,
    ],
    compiler_params=pltpu.CompilerParams(use_tc_tiling_on_sc=False),
)
def _gather_kernel(pos_hbm, rot_hbm, idx_hbm, pos_out, rot_out,
                   idx_v, pos_v, rot_v, sem_p, sem_r):
    wid = lax.axis_index("s") * _NC + lax.axis_index("c")
    pltpu.sync_copy(idx_hbm.at[pl.ds(wid * _NCHUNK, _NCHUNK)], idx_v)
    copies = []
    for j in range(_NCHUNK):
        for r in range(3):
            copies.append(pltpu.async_copy(
                pos_hbm.at[r].at[idx_v.at[j]], pos_v.at[j, r], sem_p))
        for r in range(6):
            copies.append(pltpu.async_copy(
                rot_hbm.at[r].at[idx_v.at[j]], rot_v.at[j, r], sem_r))
    for c in copies:
        c.wait()
    blk = pl.ds(wid * _NCHUNK, _NCHUNK)
    pltpu.sync_copy(pos_v, pos_out.at[blk, pl.ds(0, 3)])
    pltpu.sync_copy(rot_v, rot_out.at[blk, pl.ds(0, 6)])


def kernel(positions, rotations, camera_indices):
    idx2d = camera_indices.astype(jnp.int32).reshape(BATCH // _CHUNK, _CHUNK)
    pos3, rot3 = _gather_kernel(positions.T, rotations.T, idx2d)
    pos = pos3.transpose(0, 2, 1).reshape(BATCH, 4)[:, :3]
    rot = rot3.transpose(0, 2, 1).reshape(BATCH, 8)[:, :6]
    return pos, rot
